# Initial kernel scaffold; baseline (speedup 1.0000x reference)
import jax
import jax.numpy as jnp
from jax.experimental import pallas as pl


def kernel(hidden_states, W_up, b_up, W_down, b_down, W_q, keys, down_embed, up_embed):
    raise NotImplementedError("write your pallas kernel here")


# speedup vs baseline: 11.3557x; 11.3557x over previous
import functools

import jax
import jax.numpy as jnp
from jax import lax
from jax.experimental import pallas as pl
from jax.experimental.pallas import tpu as pltpu
from jax.experimental.pallas import tpu_sc as plsc

_K = 8
_NK = 64
_H = 4
_T = 2048
_DM = 1024
_DCD = 2048
_DPE = 512
_NE = 4096
_BT = 256
_NC = 2
_NS = 16
_NW = _NC * _NS
_TPW = _T // _NW
_C = 16
_NEG = float("-inf")


def _front_body(x_ref, wup_ref, bup_ref, wdown_ref, bdown_ref, wq_ref,
                kmt_ref, det_ref, logits_ref, gidx_ref, ss_ref):
    pid = pl.program_id(0)
    x = x_ref[...]
    h1 = jnp.dot(x, wup_ref[...], preferred_element_type=jnp.float32)
    h1 = h1 + bup_ref[...]
    h1 = h1 * (1.0 / (1.0 + jnp.exp(-h1)))
    h = jnp.dot(h1, wdown_ref[...], preferred_element_type=jnp.float32)
    h = h + bdown_ref[...]
    logits_ref[...] = jnp.dot(h, det_ref[...], preferred_element_type=jnp.float32)
    q = jnp.dot(h, wq_ref[...], preferred_element_type=jnp.float32)
    simt = lax.dot_general(kmt_ref[...], q, (((1,), (1,)), ((), ())),
                           preferred_element_type=jnp.float32)

    riota = lax.broadcasted_iota(jnp.int32, (_NK, _BT), 0)
    tcol = pid * _BT + lax.broadcasted_iota(jnp.int32, (1, _BT), 1)
    row32 = lax.broadcasted_iota(jnp.int32, (4 * _K, _BT), 0)

    def top8(s):
        vals, poss = [], []
        for _ in range(_K):
            m = jnp.max(s, axis=0, keepdims=True)
            am = jnp.min(jnp.where(s == m, riota, _NK), axis=0, keepdims=True)
            vals.append(m)
            poss.append(am)
            s = jnp.where(riota == am, _NEG, s)
        return vals, poss

    out_ss = jnp.zeros((4 * _K, _BT), jnp.float32)
    out_gi = jnp.zeros((4 * _K, _BT), jnp.int32)
    ra = riota // _K
    rb = riota % _K
    for hh in range(_H):
        xv, xp = top8(simt[hh * _NK:(hh + 1) * _NK, :])
        yv, yp = top8(simt[_H * _NK + hh * _NK:_H * _NK + (hh + 1) * _NK, :])
        asc = jnp.zeros((_NK, _BT), jnp.float32)
        aidx = jnp.zeros((_NK, _BT), jnp.int32)
        for a in range(_K):
            asc = asc + jnp.where(ra == a, xv[a], 0.0)
            aidx = aidx + jnp.where(ra == a, xp[a] * _NK, 0)
        for b in range(_K):
            asc = asc + jnp.where(rb == b, yv[b], 0.0)
            aidx = aidx + jnp.where(rb == b, yp[b], 0)
        scs, eids = [], []
        for _ in range(_K):
            m = jnp.max(asc, axis=0, keepdims=True)
            am = jnp.min(jnp.where(asc == m, riota, _NK), axis=0, keepdims=True)
            sel = riota == am
            eids.append(jnp.sum(jnp.where(sel, aidx, 0), axis=0, keepdims=True))
            scs.append(m)
            asc = jnp.where(sel, _NEG, asc)
        es = [jnp.exp(v - scs[0]) for v in scs]
        tot = es[0]
        for e in es[1:]:
            tot = tot + e
        inv = 1.0 / tot
        for k in range(_K):
            r = hh * _K + k
            out_ss = jnp.where(row32 == r, es[k] * inv, out_ss)
            out_gi = jnp.where(row32 == r, eids[k] + tcol * _NE, out_gi)
    ss_ref[...] = jnp.transpose(out_ss)
    gidx_ref[...] = jnp.transpose(out_gi)


def _combine_body(wsp_ref, ue_ref, out_ref):
    out_ref[...] = jnp.dot(wsp_ref[...], ue_ref[...],
                           preferred_element_type=jnp.float32)


def _sc_body(logits_hbm, gidx_hbm, ss_hbm, wsp_hbm, gi_v, ss_v, xg_v,
             chunk_v, sem):
    cid = lax.axis_index("c")
    sid = lax.axis_index("s")
    wid = sid * _NC + cid
    base = wid * _TPW
    nsel = 4 * _K * _C

    zero16 = jnp.zeros((16,), jnp.float32)

    def zbody(i, carry):
        chunk_v[pl.ds(i * 16, 16)] = zero16
        return carry

    lax.fori_loop(0, _C * _NE // 16, zbody, 0)

    for ci in range(_TPW // _C):
        t0 = base + ci * _C
        pltpu.sync_copy(gidx_hbm.at[pl.ds(t0 * 4 * _K, nsel)], gi_v)
        pltpu.sync_copy(ss_hbm.at[pl.ds(t0 * 4 * _K, nsel)], ss_v)
        pltpu.async_copy(logits_hbm.at[gi_v], xg_v, sem).wait()
        for j in range(nsel // 16):
            sl = pl.ds(j * 16, 16)
            z = xg_v[sl] * ss_v[sl]
            w = z * (1.0 / (1.0 + jnp.exp(-z)))
            li = gi_v[sl] - t0 * _NE
            plsc.addupdate_scatter(chunk_v, [li], w)
        pltpu.sync_copy(chunk_v, wsp_hbm.at[pl.ds(t0 * _NE, _C * _NE)])
        for j in range(nsel // 16):
            li = gi_v[pl.ds(j * 16, 16)] - t0 * _NE
            plsc.store_scatter(chunk_v, [li], zero16)


def _routing_sc(logits, gidx, ss):
    mesh = plsc.VectorSubcoreMesh(core_axis_name="c", subcore_axis_name="s")
    f = pl.kernel(
        _sc_body,
        out_type=jax.ShapeDtypeStruct((_T * _NE,), jnp.float32),
        mesh=mesh,
        scratch_types=[
            pltpu.VMEM((4 * _K * _C,), jnp.int32),
            pltpu.VMEM((4 * _K * _C,), jnp.float32),
            pltpu.VMEM((4 * _K * _C,), jnp.float32),
            pltpu.VMEM((_C * _NE,), jnp.float32),
            pltpu.SemaphoreType.DMA,
        ],
        compiler_params=pltpu.CompilerParams(needs_layout_passes=False),
    )
    return f(logits.reshape(_T * _NE), gidx.reshape(_T * 4 * _K),
             ss.reshape(_T * 4 * _K))


def kernel(hidden_states, W_up, b_up, W_down, b_down, W_q, keys, down_embed,
           up_embed):
    x = hidden_states.reshape(_T, _DM)
    kk = keys.transpose(2, 0, 1, 3).reshape(2 * _H, _NK, _NK)
    eye8 = jnp.eye(2 * _H, dtype=keys.dtype)
    kmt = jnp.einsum('gkn,gG->gkGn', kk, eye8).reshape(2 * _H * _NK,
                                                       2 * _H * _NK)

    grid = _T // _BT
    logits, gidx, ss = pl.pallas_call(
        _front_body,
        grid=(grid,),
        in_specs=[
            pl.BlockSpec((_BT, _DM), lambda i: (i, 0)),
            pl.BlockSpec((_DM, _DCD), lambda i: (0, 0)),
            pl.BlockSpec((1, _DCD), lambda i: (0, 0)),
            pl.BlockSpec((_DCD, _DPE), lambda i: (0, 0)),
            pl.BlockSpec((1, _DPE), lambda i: (0, 0)),
            pl.BlockSpec((_DPE, _DPE), lambda i: (0, 0)),
            pl.BlockSpec((_DPE, _DPE), lambda i: (0, 0)),
            pl.BlockSpec((_DPE, _NE), lambda i: (0, 0)),
        ],
        out_specs=[
            pl.BlockSpec((_BT, _NE), lambda i: (i, 0)),
            pl.BlockSpec((_BT, 4 * _K), lambda i: (i, 0)),
            pl.BlockSpec((_BT, 4 * _K), lambda i: (i, 0)),
        ],
        out_shape=[
            jax.ShapeDtypeStruct((_T, _NE), jnp.float32),
            jax.ShapeDtypeStruct((_T, 4 * _K), jnp.int32),
            jax.ShapeDtypeStruct((_T, 4 * _K), jnp.float32),
        ],
    )(x, W_up, b_up.reshape(1, _DCD), W_down, b_down.reshape(1, _DPE), W_q,
      kmt, down_embed.T)

    wsp = _routing_sc(logits, gidx, ss).reshape(_T, _NE)

    out = pl.pallas_call(
        _combine_body,
        grid=(grid,),
        in_specs=[
            pl.BlockSpec((_BT, _NE), lambda i: (i, 0)),
            pl.BlockSpec((_NE, _DM), lambda i: (0, 0)),
        ],
        out_specs=pl.BlockSpec((_BT, _DM), lambda i: (i, 0)),
        out_shape=jax.ShapeDtypeStruct((_T, _DM), jnp.float32),
    )(wsp, up_embed)

    return out.reshape(1, _T, _DM)


# speedup vs baseline: 11.5598x; 1.0180x over previous
import functools

import jax
import jax.numpy as jnp
from jax import lax
from jax.experimental import pallas as pl
from jax.experimental.pallas import tpu as pltpu
from jax.experimental.pallas import tpu_sc as plsc

_K = 8
_NK = 64
_H = 4
_T = 2048
_DM = 1024
_DCD = 2048
_DPE = 512
_NE = 4096
_BT = 256
_NC = 2
_NS = 16
_NW = _NC * _NS
_TPW = _T // _NW
_C = 16
_NEG = float("-inf")


def _front_body(x_ref, wup_ref, bup_ref, wdown_ref, bdown_ref, wq_ref,
                kmt_ref, det_ref, logits_ref, gidx_ref, ss_ref):
    pid = pl.program_id(0)
    x = x_ref[...]
    h1 = jnp.dot(x, wup_ref[...], preferred_element_type=jnp.float32)
    h1 = h1 + bup_ref[...]
    h1 = h1 * (1.0 / (1.0 + jnp.exp(-h1)))
    h = jnp.dot(h1, wdown_ref[...], preferred_element_type=jnp.float32)
    h = h + bdown_ref[...]
    logits_ref[...] = jnp.dot(h.astype(jnp.bfloat16), det_ref[...],
                              preferred_element_type=jnp.float32)
    q = jnp.dot(h, wq_ref[...], preferred_element_type=jnp.float32)
    simt = lax.dot_general(kmt_ref[...], q, (((1,), (1,)), ((), ())),
                           preferred_element_type=jnp.float32)

    riota = lax.broadcasted_iota(jnp.int32, (_NK, _BT), 0)
    tcol = pid * _BT + lax.broadcasted_iota(jnp.int32, (1, _BT), 1)
    row32 = lax.broadcasted_iota(jnp.int32, (4 * _K, _BT), 0)

    def top8(s):
        vals, poss = [], []
        for _ in range(_K):
            m = jnp.max(s, axis=0, keepdims=True)
            am = jnp.min(jnp.where(s == m, riota, _NK), axis=0, keepdims=True)
            vals.append(m)
            poss.append(am)
            s = jnp.where(riota == am, _NEG, s)
        return vals, poss

    out_ss = jnp.zeros((4 * _K, _BT), jnp.float32)
    out_gi = jnp.zeros((4 * _K, _BT), jnp.int32)
    ra = riota // _K
    rb = riota % _K
    for hh in range(_H):
        xv, xp = top8(simt[hh * _NK:(hh + 1) * _NK, :])
        yv, yp = top8(simt[_H * _NK + hh * _NK:_H * _NK + (hh + 1) * _NK, :])
        asc = jnp.zeros((_NK, _BT), jnp.float32)
        aidx = jnp.zeros((_NK, _BT), jnp.int32)
        for a in range(_K):
            asc = asc + jnp.where(ra == a, xv[a], 0.0)
            aidx = aidx + jnp.where(ra == a, xp[a] * _NK, 0)
        for b in range(_K):
            asc = asc + jnp.where(rb == b, yv[b], 0.0)
            aidx = aidx + jnp.where(rb == b, yp[b], 0)
        scs, eids = [], []
        for _ in range(_K):
            m = jnp.max(asc, axis=0, keepdims=True)
            am = jnp.min(jnp.where(asc == m, riota, _NK), axis=0, keepdims=True)
            sel = riota == am
            eids.append(jnp.sum(jnp.where(sel, aidx, 0), axis=0, keepdims=True))
            scs.append(m)
            asc = jnp.where(sel, _NEG, asc)
        es = [jnp.exp(v - scs[0]) for v in scs]
        tot = es[0]
        for e in es[1:]:
            tot = tot + e
        inv = 1.0 / tot
        for k in range(_K):
            r = hh * _K + k
            out_ss = jnp.where(row32 == r, es[k] * inv, out_ss)
            out_gi = jnp.where(row32 == r, eids[k] + tcol * _NE, out_gi)
    ss_ref[...] = jnp.transpose(out_ss)
    gidx_ref[...] = jnp.transpose(out_gi)


def _combine_body(wsp_ref, ue_ref, out_ref):
    out_ref[...] = jnp.dot(wsp_ref[...].astype(jnp.bfloat16), ue_ref[...],
                           preferred_element_type=jnp.float32)


def _sc_body(logits_hbm, gidx_hbm, ss_hbm, wsp_hbm, gi_v, ss_v, xg_v,
             chunk_v, sem):
    cid = lax.axis_index("c")
    sid = lax.axis_index("s")
    wid = sid * _NC + cid
    base = wid * _TPW
    nsel = 4 * _K * _C

    zero16 = jnp.zeros((16,), jnp.float32)

    def zbody(i, carry):
        chunk_v[pl.ds(i * 16, 16)] = zero16
        return carry

    lax.fori_loop(0, _C * _NE // 16, zbody, 0)

    for ci in range(_TPW // _C):
        t0 = base + ci * _C
        pltpu.sync_copy(gidx_hbm.at[pl.ds(t0 * 4 * _K, nsel)], gi_v)
        pltpu.sync_copy(ss_hbm.at[pl.ds(t0 * 4 * _K, nsel)], ss_v)
        pltpu.async_copy(logits_hbm.at[gi_v], xg_v, sem).wait()
        for j in range(nsel // 16):
            sl = pl.ds(j * 16, 16)
            z = xg_v[sl] * ss_v[sl]
            w = z * (1.0 / (1.0 + jnp.exp(-z)))
            li = gi_v[sl] - t0 * _NE
            plsc.addupdate_scatter(chunk_v, [li], w)
        pltpu.sync_copy(chunk_v, wsp_hbm.at[pl.ds(t0 * _NE, _C * _NE)])
        for j in range(nsel // 16):
            li = gi_v[pl.ds(j * 16, 16)] - t0 * _NE
            plsc.store_scatter(chunk_v, [li], zero16)


def _routing_sc(logits, gidx, ss):
    mesh = plsc.VectorSubcoreMesh(core_axis_name="c", subcore_axis_name="s")
    f = pl.kernel(
        _sc_body,
        out_type=jax.ShapeDtypeStruct((_T * _NE,), jnp.float32),
        mesh=mesh,
        scratch_types=[
            pltpu.VMEM((4 * _K * _C,), jnp.int32),
            pltpu.VMEM((4 * _K * _C,), jnp.float32),
            pltpu.VMEM((4 * _K * _C,), jnp.float32),
            pltpu.VMEM((_C * _NE,), jnp.float32),
            pltpu.SemaphoreType.DMA,
        ],
        compiler_params=pltpu.CompilerParams(needs_layout_passes=False),
    )
    return f(logits.reshape(_T * _NE), gidx.reshape(_T * 4 * _K),
             ss.reshape(_T * 4 * _K))


def kernel(hidden_states, W_up, b_up, W_down, b_down, W_q, keys, down_embed,
           up_embed):
    x = hidden_states.reshape(_T, _DM)
    kk = keys.transpose(2, 0, 1, 3).reshape(2 * _H, _NK, _NK)
    eye8 = jnp.eye(2 * _H, dtype=keys.dtype)
    kmt = jnp.einsum('gkn,gG->gkGn', kk, eye8).reshape(2 * _H * _NK,
                                                       2 * _H * _NK)

    grid = _T // _BT
    logits, gidx, ss = pl.pallas_call(
        _front_body,
        grid=(grid,),
        in_specs=[
            pl.BlockSpec((_BT, _DM), lambda i: (i, 0)),
            pl.BlockSpec((_DM, _DCD), lambda i: (0, 0)),
            pl.BlockSpec((1, _DCD), lambda i: (0, 0)),
            pl.BlockSpec((_DCD, _DPE), lambda i: (0, 0)),
            pl.BlockSpec((1, _DPE), lambda i: (0, 0)),
            pl.BlockSpec((_DPE, _DPE), lambda i: (0, 0)),
            pl.BlockSpec((_DPE, _DPE), lambda i: (0, 0)),
            pl.BlockSpec((_DPE, _NE), lambda i: (0, 0)),
        ],
        out_specs=[
            pl.BlockSpec((_BT, _NE), lambda i: (i, 0)),
            pl.BlockSpec((_BT, 4 * _K), lambda i: (i, 0)),
            pl.BlockSpec((_BT, 4 * _K), lambda i: (i, 0)),
        ],
        out_shape=[
            jax.ShapeDtypeStruct((_T, _NE), jnp.float32),
            jax.ShapeDtypeStruct((_T, 4 * _K), jnp.int32),
            jax.ShapeDtypeStruct((_T, 4 * _K), jnp.float32),
        ],
    )(x, W_up, b_up.reshape(1, _DCD), W_down, b_down.reshape(1, _DPE), W_q,
      kmt, down_embed.T.astype(jnp.bfloat16))

    wsp = _routing_sc(logits, gidx, ss).reshape(_T, _NE)

    out = pl.pallas_call(
        _combine_body,
        grid=(grid,),
        in_specs=[
            pl.BlockSpec((_BT, _NE), lambda i: (i, 0)),
            pl.BlockSpec((_NE, _DM), lambda i: (0, 0)),
        ],
        out_specs=pl.BlockSpec((_BT, _DM), lambda i: (i, 0)),
        out_shape=jax.ShapeDtypeStruct((_T, _DM), jnp.float32),
    )(wsp, up_embed.astype(jnp.bfloat16))

    return out.reshape(1, _T, _DM)


# speedup vs baseline: 15.0448x; 1.3015x over previous
import jax
import jax.numpy as jnp
from jax import lax
from jax.experimental import pallas as pl
from jax.experimental.pallas import tpu as pltpu
from jax.experimental.pallas import tpu_sc as plsc

_K = 8
_NK = 64
_H = 4
_T = 2048
_DM = 1024
_DCD = 2048
_DPE = 512
_NE = 4096
_G = _NE // 128
_BT = 256
_NC = 2
_NS = 16
_NW = _NC * _NS
_TPW = _T // _NW
_C = 16
_NEG = float("-inf")


def _front_body(x_ref, wup_ref, bup_ref, wdown_ref, bdown_ref, wq_ref,
                kmt_ref, det_ref, logits_ref, gidx_ref, ss_ref):
    pid = pl.program_id(0)
    x = x_ref[...]
    h1 = jnp.dot(x, wup_ref[...], preferred_element_type=jnp.float32)
    h1 = h1 + bup_ref[...]
    h1 = h1 * (1.0 / (1.0 + jnp.exp(-h1)))
    h = jnp.dot(h1, wdown_ref[...], preferred_element_type=jnp.float32)
    h = h + bdown_ref[...]
    lb = jnp.dot(h.astype(jnp.bfloat16), det_ref[...],
                 preferred_element_type=jnp.float32)
    for g in range(_G):
        logits_ref[pl.ds(g * _BT * 128, _BT * 128)] = (
            lb[:, g * 128:(g + 1) * 128].reshape(_BT * 128))
    q = jnp.dot(h, wq_ref[...], preferred_element_type=jnp.float32)
    simt = lax.dot_general(kmt_ref[...], q, (((1,), (1,)), ((), ())),
                           preferred_element_type=jnp.float32)

    riota = lax.broadcasted_iota(jnp.int32, (_NK, _BT), 0)
    tloc = lax.broadcasted_iota(jnp.int32, (1, _BT), 1)
    row32 = lax.broadcasted_iota(jnp.int32, (4 * _K, _BT), 0)

    def top8(s):
        vals, poss = [], []
        for _ in range(_K):
            m = jnp.max(s, axis=0, keepdims=True)
            am = jnp.min(jnp.where(s == m, riota, _NK), axis=0, keepdims=True)
            vals.append(m)
            poss.append(am)
            s = jnp.where(riota == am, _NEG, s)
        return vals, poss

    out_ss = jnp.zeros((4 * _K, _BT), jnp.float32)
    out_gi = jnp.zeros((4 * _K, _BT), jnp.int32)
    ra = riota // _K
    rb = riota % _K
    for hh in range(_H):
        xv, xp = top8(simt[hh * _NK:(hh + 1) * _NK, :])
        yv, yp = top8(simt[_H * _NK + hh * _NK:_H * _NK + (hh + 1) * _NK, :])
        asc = jnp.zeros((_NK, _BT), jnp.float32)
        aidx = jnp.zeros((_NK, _BT), jnp.int32)
        for a in range(_K):
            asc = asc + jnp.where(ra == a, xv[a], 0.0)
            aidx = aidx + jnp.where(ra == a, xp[a] * _NK, 0)
        for b in range(_K):
            asc = asc + jnp.where(rb == b, yv[b], 0.0)
            aidx = aidx + jnp.where(rb == b, yp[b], 0)
        scs, eids = [], []
        for _ in range(_K):
            m = jnp.max(asc, axis=0, keepdims=True)
            am = jnp.min(jnp.where(asc == m, riota, _NK), axis=0, keepdims=True)
            sel = riota == am
            eids.append(jnp.sum(jnp.where(sel, aidx, 0), axis=0, keepdims=True))
            scs.append(m)
            asc = jnp.where(sel, _NEG, asc)
        es = [jnp.exp(v - scs[0]) for v in scs]
        tot = es[0]
        for e in es[1:]:
            tot = tot + e
        inv = 1.0 / tot
        for k in range(_K):
            r = hh * _K + k
            gf = (pid * (_G * _BT * 128) + (eids[k] // 128) * (_BT * 128)
                  + tloc * 128 + (eids[k] % 128))
            out_ss = jnp.where(row32 == r, es[k] * inv, out_ss)
            out_gi = jnp.where(row32 == r, gf, out_gi)
    ss_ref[...] = jnp.transpose(out_ss)
    gidx_ref[...] = jnp.transpose(out_gi)


def _combine_body(wsp_ref, ue_ref, out_ref):
    acc = jnp.dot(wsp_ref[0].astype(jnp.bfloat16), ue_ref[0],
                  preferred_element_type=jnp.float32)
    for g in range(1, _G):
        acc = acc + jnp.dot(wsp_ref[g].astype(jnp.bfloat16), ue_ref[g],
                            preferred_element_type=jnp.float32)
    out_ref[...] = acc


def _sc_body(logits_hbm, gidx_hbm, ss_hbm, zc_hbm, wsp_hbm, gi_v, ss_v, xg_v,
             chunk_v, sem):
    cid = lax.axis_index("c")
    sid = lax.axis_index("s")
    wid = sid * _NC + cid
    base = wid * _TPW
    nsel = 4 * _K * _C

    zero16 = jnp.zeros((16,), jnp.float32)
    pltpu.sync_copy(zc_hbm, chunk_v)

    for ci in range(_TPW // _C):
        t0 = base + ci * _C
        tb = t0 & (_BT - 1)
        pltpu.sync_copy(gidx_hbm.at[pl.ds(t0 * 4 * _K, nsel)], gi_v)
        pltpu.sync_copy(ss_hbm.at[pl.ds(t0 * 4 * _K, nsel)], ss_v)
        pltpu.async_copy(logits_hbm.at[gi_v], xg_v, sem).wait()
        for j in range(nsel // 16):
            sl = pl.ds(j * 16, 16)
            z = xg_v[sl] * ss_v[sl]
            w = z * (1.0 / (1.0 + jnp.exp(-z)))
            gf = gi_v[sl]
            plsc.addupdate_scatter(
                chunk_v,
                [lax.shift_right_logical(gf, 15) & (_G - 1),
                 (lax.shift_right_logical(gf, 7) & (_BT - 1)) - tb,
                 gf & 127],
                w)
        pltpu.sync_copy(chunk_v, wsp_hbm.at[:, pl.ds(t0, _C), :])
        for j in range(nsel // 16):
            gf = gi_v[pl.ds(j * 16, 16)]
            plsc.store_scatter(
                chunk_v,
                [lax.shift_right_logical(gf, 15) & (_G - 1),
                 (lax.shift_right_logical(gf, 7) & (_BT - 1)) - tb,
                 gf & 127],
                zero16)


def _routing_sc(logits, gidx, ss):
    mesh = plsc.VectorSubcoreMesh(core_axis_name="c", subcore_axis_name="s")
    f = pl.kernel(
        _sc_body,
        out_type=jax.ShapeDtypeStruct((_G, _T, 128), jnp.float32),
        mesh=mesh,
        scratch_types=[
            pltpu.VMEM((4 * _K * _C,), jnp.int32),
            pltpu.VMEM((4 * _K * _C,), jnp.float32),
            pltpu.VMEM((4 * _K * _C,), jnp.float32),
            pltpu.VMEM((_G, _C, 128), jnp.float32),
            pltpu.SemaphoreType.DMA,
        ],
        compiler_params=pltpu.CompilerParams(needs_layout_passes=False),
    )
    return f(logits, gidx.reshape(_T * 4 * _K), ss.reshape(_T * 4 * _K),
             jnp.zeros((_G, _C, 128), jnp.float32))


def kernel(hidden_states, W_up, b_up, W_down, b_down, W_q, keys, down_embed,
           up_embed):
    x = hidden_states.reshape(_T, _DM)
    kk = keys.transpose(2, 0, 1, 3).reshape(2 * _H, _NK, _NK)
    eye8 = jnp.eye(2 * _H, dtype=keys.dtype)
    kmt = jnp.einsum('gkn,gG->gkGn', kk, eye8).reshape(2 * _H * _NK,
                                                       2 * _H * _NK)

    grid = _T // _BT
    logits, gidx, ss = pl.pallas_call(
        _front_body,
        grid=(grid,),
        in_specs=[
            pl.BlockSpec((_BT, _DM), lambda i: (i, 0)),
            pl.BlockSpec((_DM, _DCD), lambda i: (0, 0)),
            pl.BlockSpec((1, _DCD), lambda i: (0, 0)),
            pl.BlockSpec((_DCD, _DPE), lambda i: (0, 0)),
            pl.BlockSpec((1, _DPE), lambda i: (0, 0)),
            pl.BlockSpec((_DPE, _DPE), lambda i: (0, 0)),
            pl.BlockSpec((_DPE, _DPE), lambda i: (0, 0)),
            pl.BlockSpec((_DPE, _NE), lambda i: (0, 0)),
        ],
        out_specs=[
            pl.BlockSpec((_G * _BT * 128,), lambda i: (i,)),
            pl.BlockSpec((_BT, 4 * _K), lambda i: (i, 0)),
            pl.BlockSpec((_BT, 4 * _K), lambda i: (i, 0)),
        ],
        out_shape=[
            jax.ShapeDtypeStruct(((_T // _BT) * _G * _BT * 128,), jnp.float32),
            jax.ShapeDtypeStruct((_T, 4 * _K), jnp.int32),
            jax.ShapeDtypeStruct((_T, 4 * _K), jnp.float32),
        ],
    )(x, W_up, b_up.reshape(1, _DCD), W_down, b_down.reshape(1, _DPE), W_q,
      kmt, down_embed.T.astype(jnp.bfloat16))

    wsp = _routing_sc(logits, gidx, ss)

    ueb = up_embed.astype(jnp.bfloat16).reshape(_G, 128, _DM)
    out = pl.pallas_call(
        _combine_body,
        grid=(grid,),
        in_specs=[
            pl.BlockSpec((_G, _BT, 128), lambda i: (0, i, 0)),
            pl.BlockSpec((_G, 128, _DM), lambda i: (0, 0, 0)),
        ],
        out_specs=pl.BlockSpec((_BT, _DM), lambda i: (i, 0)),
        out_shape=jax.ShapeDtypeStruct((_T, _DM), jnp.float32),
    )(wsp, ueb)

    return out.reshape(1, _T, _DM)
